# baseline (speedup 1.0000x reference)
import functools

import jax
import jax.numpy as jnp
from jax import lax
from jax.experimental import pallas as pl
from jax.experimental.pallas import tpu as pltpu
from jax.experimental.pallas import tpu_sc as plsc

BATCH = 16384
EMBED_D = 64
PAIR_D = 2 * EMBED_D
HIDDEN = 256

_NC = 2
_NS = 16
_NW = _NC * _NS
_BPW = BATCH // _NW
_CHUNK = 128
_NCHUNK = _BPW // _CHUNK


def _make_sc_gather():
    mesh = plsc.VectorSubcoreMesh(core_axis_name="c", subcore_axis_name="s")

    @functools.partial(
        pl.kernel,
        mesh=mesh,
        out_type=[
            jax.ShapeDtypeStruct((BATCH, PAIR_D), jnp.float32),
            jax.ShapeDtypeStruct((BATCH, PAIR_D), jnp.float32),
        ],
        scratch_types=[
            pltpu.VMEM((_BPW,), jnp.int32),
            pltpu.VMEM((_BPW,), jnp.int32),
            pltpu.VMEM((_BPW, PAIR_D), jnp.float32),
            pltpu.SemaphoreType.DMA,
        ],
    )
    def gather_kernel(users_hbm, movies_hbm, utab_hbm, mtab_hbm,
                      uout_hbm, mout_hbm,
                      idx_v, pidx_v, rows_v, sem):
        wid = lax.axis_index("s") * _NC + lax.axis_index("c")
        base = wid * _BPW

        def gather_one(src_idx_hbm, tab_hbm, out_hbm):
            pltpu.sync_copy(src_idx_hbm.at[pl.ds(base, _BPW)], idx_v)

            @pl.loop(0, _BPW, step=16)
            def _(i):
                pidx_v[pl.ds(i, 16)] = lax.shift_right_logical(
                    idx_v[pl.ds(i, 16)], 1)

            copies = []
            for c in range(_NCHUNK):
                copies.append(pltpu.async_copy(
                    tab_hbm.at[pidx_v.at[pl.ds(c * _CHUNK, _CHUNK)]],
                    rows_v.at[pl.ds(c * _CHUNK, _CHUNK), :],
                    sem,
                ))
            for cp in copies:
                cp.wait()

            pltpu.sync_copy(rows_v, out_hbm.at[pl.ds(base, _BPW)])

        gather_one(users_hbm, utab_hbm, uout_hbm)
        gather_one(movies_hbm, mtab_hbm, mout_hbm)

    return gather_kernel


_sc_gather = _make_sc_gather()

_ROWS_BLK = 1024


def _mlp_body(up_ref, mp_ref, uid_ref, mid_ref, w1_ref, w2_ref, b_ref,
              o_ref):
    up = up_ref[...]
    mp = mp_ref[...]
    usel = (uid_ref[...] & 1) == 1
    msel = (mid_ref[...] & 1) == 1
    u = jnp.where(usel, up[:, EMBED_D:], up[:, :EMBED_D])
    m = jnp.where(msel, mp[:, EMBED_D:], mp[:, :EMBED_D])
    acc = jnp.dot(u, w1_ref[...], preferred_element_type=jnp.float32)
    acc = acc + jnp.dot(m, w2_ref[...], preferred_element_type=jnp.float32)
    acc = acc + b_ref[...]
    o_ref[...] = jnp.maximum(acc, 0.0)


def _mlp(u_pairs, m_pairs, users, movies, W, b):
    w1 = W[:EMBED_D]
    w2 = W[EMBED_D:]
    b2 = b.reshape(1, HIDDEN)
    u2 = users.reshape(BATCH, 1)
    m2 = movies.reshape(BATCH, 1)
    grid = (BATCH // _ROWS_BLK,)
    return pl.pallas_call(
        _mlp_body,
        grid=grid,
        in_specs=[
            pl.BlockSpec((_ROWS_BLK, PAIR_D), lambda i: (i, 0)),
            pl.BlockSpec((_ROWS_BLK, PAIR_D), lambda i: (i, 0)),
            pl.BlockSpec((_ROWS_BLK, 1), lambda i: (i, 0)),
            pl.BlockSpec((_ROWS_BLK, 1), lambda i: (i, 0)),
            pl.BlockSpec((EMBED_D, HIDDEN), lambda i: (0, 0)),
            pl.BlockSpec((EMBED_D, HIDDEN), lambda i: (0, 0)),
            pl.BlockSpec((1, HIDDEN), lambda i: (0, 0)),
        ],
        out_specs=pl.BlockSpec((_ROWS_BLK, HIDDEN), lambda i: (i, 0)),
        out_shape=jax.ShapeDtypeStruct((BATCH, HIDDEN), jnp.float32),
    )(u_pairs, m_pairs, u2, m2, w1, w2, b2)


@jax.jit
def kernel(users, movies, user_table, movie_table, W, b):
    users = users.astype(jnp.int32)
    movies = movies.astype(jnp.int32)
    ut2 = user_table.reshape(-1, PAIR_D)
    mt2 = movie_table.reshape(-1, PAIR_D)
    u_pairs, m_pairs = _sc_gather(users, movies, ut2, mt2)
    return _mlp(u_pairs, m_pairs, users, movies, W, b)


# speedup vs baseline: 1.6645x; 1.6645x over previous
import functools

import jax
import jax.numpy as jnp
from jax import lax
from jax.experimental import pallas as pl
from jax.experimental.pallas import tpu as pltpu
from jax.experimental.pallas import tpu_sc as plsc

BATCH = 16384
EMBED_D = 64
PAIR_D = 2 * EMBED_D
HIDDEN = 256

_NC = 2
_NS = 16
_NW = _NC * _NS
_BPW = BATCH // _NW
_CHUNK = 128
_NCHUNK = _BPW // _CHUNK

_TCOLS = 4096
_HALF = _TCOLS // 2


def _relayout_body(in_ref, out_ref):
    x = in_ref[...]
    ta = lax.transpose(x[:, :_HALF], (1, 0))
    tb = lax.transpose(x[:, _HALF:], (1, 0))
    out_ref[...] = jnp.concatenate([ta, tb], axis=1)


def _relayout(table_t, n_rows):
    grid_n = -(-n_rows // _TCOLS)
    return pl.pallas_call(
        _relayout_body,
        grid=(grid_n,),
        in_specs=[pl.BlockSpec((EMBED_D, _TCOLS), lambda i: (0, i))],
        out_specs=pl.BlockSpec((_HALF, PAIR_D), lambda i: (i, 0)),
        out_shape=jax.ShapeDtypeStruct((grid_n * _HALF, PAIR_D),
                                       jnp.float32),
    )(table_t)


def _make_sc_gather(pair_rows):
    mesh = plsc.VectorSubcoreMesh(core_axis_name="c", subcore_axis_name="s")

    @functools.partial(
        pl.kernel,
        mesh=mesh,
        out_type=jax.ShapeDtypeStruct((BATCH, PAIR_D), jnp.float32),
        scratch_types=[
            pltpu.VMEM((_BPW,), jnp.int32),
            pltpu.VMEM((_BPW,), jnp.int32),
            pltpu.VMEM((_BPW, PAIR_D), jnp.float32),
            pltpu.SemaphoreType.DMA,
        ],
    )
    def gather_kernel(idx_hbm, tab_hbm, out_hbm, idx_v, pidx_v, rows_v, sem):
        wid = lax.axis_index("s") * _NC + lax.axis_index("c")
        base = wid * _BPW

        pltpu.sync_copy(idx_hbm.at[pl.ds(base, _BPW)], idx_v)

        @pl.loop(0, _BPW, step=16)
        def _(i):
            v = idx_v[pl.ds(i, 16)]
            hi = lax.shift_left(lax.shift_right_logical(v, 12), 11)
            lo = lax.bitwise_and(v, 2047)
            pidx_v[pl.ds(i, 16)] = lax.bitwise_or(hi, lo)

        copies = []
        for c in range(_NCHUNK):
            copies.append(pltpu.async_copy(
                tab_hbm.at[pidx_v.at[pl.ds(c * _CHUNK, _CHUNK)]],
                rows_v.at[pl.ds(c * _CHUNK, _CHUNK), :],
                sem,
            ))
        for cp in copies:
            cp.wait()

        pltpu.sync_copy(rows_v, out_hbm.at[pl.ds(base, _BPW)])

    return gather_kernel


_USER_PAD = -(-1000000 // _TCOLS) * _HALF
_MOVIE_PAD = -(-100000 // _TCOLS) * _HALF
_gather_user = _make_sc_gather(_USER_PAD)
_gather_movie = _make_sc_gather(_MOVIE_PAD)

_ROWS_BLK = 1024


def _mlp_body(up_ref, mp_ref, uid_ref, mid_ref, w1_ref, w2_ref, b_ref,
              o_ref):
    up = up_ref[...]
    mp = mp_ref[...]
    usel = (lax.shift_right_logical(uid_ref[...], 11) & 1) == 1
    msel = (lax.shift_right_logical(mid_ref[...], 11) & 1) == 1
    u = jnp.where(usel, up[:, EMBED_D:], up[:, :EMBED_D])
    m = jnp.where(msel, mp[:, EMBED_D:], mp[:, :EMBED_D])
    acc = jnp.dot(u, w1_ref[...], preferred_element_type=jnp.float32)
    acc = acc + jnp.dot(m, w2_ref[...], preferred_element_type=jnp.float32)
    acc = acc + b_ref[...]
    o_ref[...] = jnp.maximum(acc, 0.0)


def _mlp(u_pairs, m_pairs, users, movies, W, b):
    w1 = W[:EMBED_D]
    w2 = W[EMBED_D:]
    b2 = b.reshape(1, HIDDEN)
    u2 = users.reshape(BATCH, 1)
    m2 = movies.reshape(BATCH, 1)
    grid = (BATCH // _ROWS_BLK,)
    return pl.pallas_call(
        _mlp_body,
        grid=grid,
        in_specs=[
            pl.BlockSpec((_ROWS_BLK, PAIR_D), lambda i: (i, 0)),
            pl.BlockSpec((_ROWS_BLK, PAIR_D), lambda i: (i, 0)),
            pl.BlockSpec((_ROWS_BLK, 1), lambda i: (i, 0)),
            pl.BlockSpec((_ROWS_BLK, 1), lambda i: (i, 0)),
            pl.BlockSpec((EMBED_D, HIDDEN), lambda i: (0, 0)),
            pl.BlockSpec((EMBED_D, HIDDEN), lambda i: (0, 0)),
            pl.BlockSpec((1, HIDDEN), lambda i: (0, 0)),
        ],
        out_specs=pl.BlockSpec((_ROWS_BLK, HIDDEN), lambda i: (i, 0)),
        out_shape=jax.ShapeDtypeStruct((BATCH, HIDDEN), jnp.float32),
    )(u_pairs, m_pairs, u2, m2, w1, w2, b2)


@jax.jit
def kernel(users, movies, user_table, movie_table, W, b):
    users = users.astype(jnp.int32)
    movies = movies.astype(jnp.int32)
    mt2 = _relayout(movie_table.T, movie_table.shape[0])
    m_pairs = _gather_movie(movies, mt2)
    ut2 = _relayout(user_table.T, user_table.shape[0])
    u_pairs = _gather_user(users, ut2)
    return _mlp(u_pairs, m_pairs, users, movies, W, b)


# speedup vs baseline: 2.0769x; 1.2478x over previous
import functools

import jax
import jax.numpy as jnp
from jax import lax
from jax.experimental import pallas as pl
from jax.experimental.pallas import tpu as pltpu
from jax.experimental.pallas import tpu_sc as plsc

BATCH = 16384
EMBED_D = 64
PAIR_D = 2 * EMBED_D
HIDDEN = 256

_NC = 2
_NS = 16
_NW = _NC * _NS
_BPW = BATCH // _NW
_CHUNK = 128
_NCHUNK = _BPW // _CHUNK

_TCOLS = 8192
_HALF = _TCOLS // 2
_BLK_SHIFT = _TCOLS.bit_length() - 1
_HALF_SHIFT = _BLK_SHIFT - 1
_HALF_MASK = _HALF - 1


def _relayout_body(in_ref, out_ref):
    x = in_ref[...]
    ta = lax.transpose(x[:, :_HALF], (1, 0))
    tb = lax.transpose(x[:, _HALF:], (1, 0))
    out_ref[...] = jnp.concatenate([ta, tb], axis=1)


def _relayout(table_t, n_rows):
    grid_n = -(-n_rows // _TCOLS)
    return pl.pallas_call(
        _relayout_body,
        grid=(grid_n,),
        in_specs=[pl.BlockSpec((EMBED_D, _TCOLS), lambda i: (0, i))],
        out_specs=pl.BlockSpec((_HALF, PAIR_D), lambda i: (i, 0)),
        out_shape=jax.ShapeDtypeStruct((grid_n * _HALF, PAIR_D),
                                       jnp.float32),
    )(table_t)


def _make_sc_gather(pair_rows):
    mesh = plsc.VectorSubcoreMesh(core_axis_name="c", subcore_axis_name="s")

    @functools.partial(
        pl.kernel,
        mesh=mesh,
        out_type=jax.ShapeDtypeStruct((BATCH, PAIR_D), jnp.float32),
        scratch_types=[
            pltpu.VMEM((_BPW,), jnp.int32),
            pltpu.VMEM((_BPW,), jnp.int32),
            pltpu.VMEM((_BPW, PAIR_D), jnp.float32),
            pltpu.SemaphoreType.DMA,
        ],
    )
    def gather_kernel(idx_hbm, tab_hbm, out_hbm, idx_v, pidx_v, rows_v, sem):
        wid = lax.axis_index("s") * _NC + lax.axis_index("c")
        base = wid * _BPW

        pltpu.sync_copy(idx_hbm.at[pl.ds(base, _BPW)], idx_v)

        @pl.loop(0, _BPW, step=16)
        def _(i):
            v = idx_v[pl.ds(i, 16)]
            hi = lax.shift_left(
                lax.shift_right_logical(v, _BLK_SHIFT), _HALF_SHIFT)
            lo = lax.bitwise_and(v, _HALF_MASK)
            pidx_v[pl.ds(i, 16)] = lax.bitwise_or(hi, lo)

        copies = []
        for c in range(_NCHUNK):
            copies.append(pltpu.async_copy(
                tab_hbm.at[pidx_v.at[pl.ds(c * _CHUNK, _CHUNK)]],
                rows_v.at[pl.ds(c * _CHUNK, _CHUNK), :],
                sem,
            ))
        for cp in copies:
            cp.wait()

        pltpu.sync_copy(rows_v, out_hbm.at[pl.ds(base, _BPW)])

    return gather_kernel


_USER_PAD = -(-1000000 // _TCOLS) * _HALF
_MOVIE_PAD = -(-100000 // _TCOLS) * _HALF
_gather_user = _make_sc_gather(_USER_PAD)
_gather_movie = _make_sc_gather(_MOVIE_PAD)

_ROWS_BLK = 2048


def _mlp_body(up_ref, mp_ref, ids_ref, w1_ref, w2_ref, b_ref, o_ref):
    up = up_ref[...]
    mp = mp_ref[...]
    uid = ids_ref[:, 0:1]
    mid = ids_ref[:, 1:2]
    usel = (lax.shift_right_logical(uid, _HALF_SHIFT) & 1) == 1
    msel = (lax.shift_right_logical(mid, _HALF_SHIFT) & 1) == 1
    u = jnp.where(usel, up[:, EMBED_D:], up[:, :EMBED_D])
    m = jnp.where(msel, mp[:, EMBED_D:], mp[:, :EMBED_D])
    acc = jnp.dot(u, w1_ref[...], preferred_element_type=jnp.float32)
    acc = acc + jnp.dot(m, w2_ref[...], preferred_element_type=jnp.float32)
    acc = acc + b_ref[...]
    o_ref[...] = jnp.maximum(acc, 0.0)


def _mlp(u_pairs, m_pairs, users, movies, W, b):
    w1 = W[:EMBED_D]
    w2 = W[EMBED_D:]
    b2 = b.reshape(1, HIDDEN)
    ids = jnp.stack([users, movies], axis=1)
    grid = (BATCH // _ROWS_BLK,)
    return pl.pallas_call(
        _mlp_body,
        grid=grid,
        in_specs=[
            pl.BlockSpec((_ROWS_BLK, PAIR_D), lambda i: (i, 0)),
            pl.BlockSpec((_ROWS_BLK, PAIR_D), lambda i: (i, 0)),
            pl.BlockSpec((_ROWS_BLK, 2), lambda i: (i, 0)),
            pl.BlockSpec((EMBED_D, HIDDEN), lambda i: (0, 0)),
            pl.BlockSpec((EMBED_D, HIDDEN), lambda i: (0, 0)),
            pl.BlockSpec((1, HIDDEN), lambda i: (0, 0)),
        ],
        out_specs=pl.BlockSpec((_ROWS_BLK, HIDDEN), lambda i: (i, 0)),
        out_shape=jax.ShapeDtypeStruct((BATCH, HIDDEN), jnp.float32),
    )(u_pairs, m_pairs, ids, w1, w2, b2)


@jax.jit
def kernel(users, movies, user_table, movie_table, W, b):
    users = users.astype(jnp.int32)
    movies = movies.astype(jnp.int32)
    mt2 = _relayout(movie_table.T, movie_table.shape[0])
    m_pairs = _gather_movie(movies, mt2)
    ut2 = _relayout(user_table.T, user_table.shape[0])
    u_pairs = _gather_user(users, ut2)
    return _mlp(u_pairs, m_pairs, users, movies, W, b)


# speedup vs baseline: 2.3171x; 1.1157x over previous
import functools

import jax
import jax.numpy as jnp
from jax import lax
from jax.experimental import pallas as pl
from jax.experimental.pallas import tpu as pltpu
from jax.experimental.pallas import tpu_sc as plsc

BATCH = 16384
EMBED_D = 64
PAIR_D = 2 * EMBED_D
HIDDEN = 256

_NC = 2
_NS = 16
_NW = _NC * _NS
_BPW = BATCH // _NW
_CHUNK = 128
_NCHUNK = _BPW // _CHUNK

_TCOLS = 16384
_HALF = _TCOLS // 2
_BLK_SHIFT = _TCOLS.bit_length() - 1
_HALF_SHIFT = _BLK_SHIFT - 1
_HALF_MASK = _HALF - 1


def _relayout_body(in_ref, out_ref):
    x = in_ref[...]
    ta = lax.transpose(x[:, :_HALF], (1, 0))
    tb = lax.transpose(x[:, _HALF:], (1, 0))
    out_ref[...] = jnp.concatenate([ta, tb], axis=1)


def _relayout(table_t, n_rows):
    grid_n = -(-n_rows // _TCOLS)
    return pl.pallas_call(
        _relayout_body,
        grid=(grid_n,),
        in_specs=[pl.BlockSpec((EMBED_D, _TCOLS), lambda i: (0, i))],
        out_specs=pl.BlockSpec((_HALF, PAIR_D), lambda i: (i, 0)),
        out_shape=jax.ShapeDtypeStruct((grid_n * _HALF, PAIR_D),
                                       jnp.float32),
    )(table_t)


def _make_sc_gather(pair_rows):
    mesh = plsc.VectorSubcoreMesh(core_axis_name="c", subcore_axis_name="s")

    @functools.partial(
        pl.kernel,
        mesh=mesh,
        out_type=jax.ShapeDtypeStruct((BATCH, PAIR_D), jnp.float32),
        scratch_types=[
            pltpu.VMEM((_BPW,), jnp.int32),
            pltpu.VMEM((_BPW,), jnp.int32),
            pltpu.VMEM((_BPW, PAIR_D), jnp.float32),
            pltpu.SemaphoreType.DMA,
        ],
    )
    def gather_kernel(idx_hbm, tab_hbm, out_hbm, idx_v, pidx_v, rows_v, sem):
        wid = lax.axis_index("s") * _NC + lax.axis_index("c")
        base = wid * _BPW

        pltpu.sync_copy(idx_hbm.at[pl.ds(base, _BPW)], idx_v)

        @pl.loop(0, _BPW, step=16)
        def _(i):
            v = idx_v[pl.ds(i, 16)]
            hi = lax.shift_left(
                lax.shift_right_logical(v, _BLK_SHIFT), _HALF_SHIFT)
            lo = lax.bitwise_and(v, _HALF_MASK)
            pidx_v[pl.ds(i, 16)] = lax.bitwise_or(hi, lo)

        copies = []
        for c in range(_NCHUNK):
            copies.append(pltpu.async_copy(
                tab_hbm.at[pidx_v.at[pl.ds(c * _CHUNK, _CHUNK)]],
                rows_v.at[pl.ds(c * _CHUNK, _CHUNK), :],
                sem,
            ))
        for cp in copies:
            cp.wait()

        pltpu.sync_copy(rows_v, out_hbm.at[pl.ds(base, _BPW)])

    return gather_kernel


_USER_PAD = -(-1000000 // _TCOLS) * _HALF
_MOVIE_PAD = -(-100000 // _TCOLS) * _HALF
_gather_user = _make_sc_gather(_USER_PAD)
_gather_movie = _make_sc_gather(_MOVIE_PAD)

_ROWS_BLK = 2048


def _mlp_body(up_ref, mp_ref, ids_ref, w1_ref, w2_ref, b_ref, o_ref):
    up = up_ref[...]
    mp = mp_ref[...]
    uid = ids_ref[:, 0:1]
    mid = ids_ref[:, 1:2]
    usel = (lax.shift_right_logical(uid, _HALF_SHIFT) & 1) == 1
    msel = (lax.shift_right_logical(mid, _HALF_SHIFT) & 1) == 1
    u = jnp.where(usel, up[:, EMBED_D:], up[:, :EMBED_D])
    m = jnp.where(msel, mp[:, EMBED_D:], mp[:, :EMBED_D])
    acc = jnp.dot(u, w1_ref[...], preferred_element_type=jnp.float32)
    acc = acc + jnp.dot(m, w2_ref[...], preferred_element_type=jnp.float32)
    acc = acc + b_ref[...]
    o_ref[...] = jnp.maximum(acc, 0.0)


def _mlp(u_pairs, m_pairs, users, movies, W, b):
    w1 = W[:EMBED_D]
    w2 = W[EMBED_D:]
    b2 = b.reshape(1, HIDDEN)
    ids = jnp.stack([users, movies], axis=1)
    grid = (BATCH // _ROWS_BLK,)
    return pl.pallas_call(
        _mlp_body,
        grid=grid,
        in_specs=[
            pl.BlockSpec((_ROWS_BLK, PAIR_D), lambda i: (i, 0)),
            pl.BlockSpec((_ROWS_BLK, PAIR_D), lambda i: (i, 0)),
            pl.BlockSpec((_ROWS_BLK, 2), lambda i: (i, 0)),
            pl.BlockSpec((EMBED_D, HIDDEN), lambda i: (0, 0)),
            pl.BlockSpec((EMBED_D, HIDDEN), lambda i: (0, 0)),
            pl.BlockSpec((1, HIDDEN), lambda i: (0, 0)),
        ],
        out_specs=pl.BlockSpec((_ROWS_BLK, HIDDEN), lambda i: (i, 0)),
        out_shape=jax.ShapeDtypeStruct((BATCH, HIDDEN), jnp.float32),
    )(u_pairs, m_pairs, ids, w1, w2, b2)


@jax.jit
def kernel(users, movies, user_table, movie_table, W, b):
    users = users.astype(jnp.int32)
    movies = movies.astype(jnp.int32)
    mt2 = _relayout(movie_table.T, movie_table.shape[0])
    m_pairs = _gather_movie(movies, mt2)
    ut2 = _relayout(user_table.T, user_table.shape[0])
    u_pairs = _gather_user(users, ut2)
    return _mlp(u_pairs, m_pairs, users, movies, W, b)
